# Initial kernel scaffold; baseline (speedup 1.0000x reference)
import jax
import jax.numpy as jnp
from jax.experimental import pallas as pl


def kernel(x_feat, edge_attr, W_atom, W_score, W_c, W_bb, W_cb, W_bc, W_cc, W_inter, W_out, edge_index):
    raise NotImplementedError("write your pallas kernel here")


# speedup vs baseline: 10.1239x; 10.1239x over previous
import functools

import jax
import jax.numpy as jnp
from jax import lax
from jax.experimental import pallas as pl

N = 10000
B = 100
NP = 100
E = 160000
EP = 1600
K = 4
D = 128
G = 10
NB = B // G

_HI = lax.Precision.HIGHEST


def _dotT(a, b):
    return lax.dot_general(a, b, (((0,), (0,)), ((), ())),
                           preferred_element_type=jnp.float32, precision=_HI)


def _mm(a, b):
    return lax.dot_general(a, b, (((1,), (0,)), ((), ())),
                           preferred_element_type=jnp.float32, precision=_HI)


def _body(src_ref, dst_ref, ea_ref, xf_ref, Wa, Ws, Wc, Wbb, Wcb, Wi, Wo,
          out_ref):
    srcs = src_ref[0]
    dsts = dst_ref[0]
    eas = ea_ref[0]
    Wio = _mm(Wi[...], Wo[...])
    lane = lax.broadcasted_iota(jnp.int32, (EP, NP), 1)
    for j in range(G):
        ohs = (srcs[:, j:j + 1] == lane).astype(jnp.float32)
        ohd = (dsts[:, j:j + 1] == lane).astype(jnp.float32)
        ea_j = eas[:, j:j + 1]
        A_w = _dotT(ohd, ohs * ea_j)
        A_c = _dotT(ohd, ohs)
        x = _mm(xf_ref[j], Wa[...])
        m = _mm(A_w, x)
        y = x + m
        sc = _mm(y, Ws[...])
        e = jnp.exp(sc - jnp.max(sc, axis=-1, keepdims=True))
        p = e / jnp.sum(e, axis=-1, keepdims=True)
        psq = p * p
        u = jnp.concatenate([x * psq[:, k:k + 1] for k in range(K)], axis=1)
        t = _mm(A_c, u)
        cents = []
        for k in range(K):
            v = x + t[:, k * D:(k + 1) * D]
            z = jnp.maximum(_mm(v, Wc[...]), 0.0)
            cents.append(_dotT(p[:, k:k + 1], z))
        cent = jnp.concatenate(cents, axis=0)
        m_cb = _mm(p, _mm(cent, Wcb[...]))
        be = jnp.maximum(_mm(y, Wbb[...]) + m_cb, 0.0)
        ge = jnp.sum(be, axis=0, keepdims=True) * (1.0 / NP)
        o = _mm(ge, Wio)
        out_ref[j] = jnp.broadcast_to(o, (1, D))


@jax.jit
def _run(srcL, dstL, ea3, xf, Wa, Ws, Wc, Wbb, Wcb, Wi, Wo):
    full = lambda i: (0, 0)
    out = pl.pallas_call(
        _body,
        grid=(NB,),
        in_specs=[
            pl.BlockSpec((1, EP, G), lambda i: (i, 0, 0)),
            pl.BlockSpec((1, EP, G), lambda i: (i, 0, 0)),
            pl.BlockSpec((1, EP, G), lambda i: (i, 0, 0)),
            pl.BlockSpec((G, NP, D), lambda i: (i, 0, 0)),
            pl.BlockSpec((D, D), full),
            pl.BlockSpec((D, K), full),
            pl.BlockSpec((D, D), full),
            pl.BlockSpec((D, D), full),
            pl.BlockSpec((D, D), full),
            pl.BlockSpec((D, D), full),
            pl.BlockSpec((D, 1), full),
        ],
        out_specs=pl.BlockSpec((G, 1, D), lambda i: (i, 0, 0)),
        out_shape=jax.ShapeDtypeStruct((B, 1, D), jnp.float32),
    )(srcL, dstL, ea3, xf, Wa, Ws, Wc, Wbb, Wcb, Wi, Wo)
    return out[:, 0, :1]


def kernel(x_feat, edge_attr, W_atom, W_score, W_c, W_bb, W_cb, W_bc, W_cc,
           W_inter, W_out, edge_index):
    del W_bc, W_cc
    goff = jnp.repeat(jnp.arange(B, dtype=jnp.int32) * NP, EP)
    srcL = (edge_index[0] - goff).reshape(NB, G, EP).transpose(0, 2, 1)
    dstL = (edge_index[1] - goff).reshape(NB, G, EP).transpose(0, 2, 1)
    ea3 = edge_attr.reshape(NB, G, EP).transpose(0, 2, 1)
    xf = x_feat.reshape(B, NP, D)
    return _run(srcL, dstL, ea3, xf, W_atom, W_score, W_c, W_bb, W_cb,
                W_inter, W_out)


# speedup vs baseline: 15.0527x; 1.4868x over previous
import functools

import jax
import jax.numpy as jnp
from jax import lax
from jax.experimental import pallas as pl
from jax.experimental.pallas import tpu as pltpu
from jax.experimental.pallas import tpu_sc as plsc

N = 10000
B = 100
NP = 100
E = 160000
EP = 1600
K = 4
D = 128
G = 10
NB = B // G

NCORE = 2
NSUB = 16
NW = NCORE * NSUB
HG = B // NCORE
HWORDS = HG * NP * NP
EH = E // NCORE
ET = 5120
ECH = ET // 128
ZCH = 31248
ZLAST = HWORDS - 15 * ZCH
PADW = 8

_HI = lax.Precision.HIGHEST


def _chunks(total):
    out, off = [], 0
    while off < total:
        sz = min(ET, total - off)
        out.append((off, sz))
        off += sz
    return out


def _sc_body(tgt_hbm, ea_hbm, zeros_hbm, aw_hbm, ac_hbm,
             idx3, eav, ones_v, zb, aw_sh, ac_sh):
    c = lax.axis_index("c")
    s = lax.axis_index("s")
    tid = c * NSUB + s

    pltpu.sync_copy(zeros_hbm, zb)
    zoff = s * ZCH

    @pl.when(s < NSUB - 1)
    def _():
        for off, sz in _chunks(ZCH):
            pltpu.sync_copy(zb.at[pl.ds(0, sz)],
                            aw_sh.at[pl.ds(zoff + off, sz)])
            pltpu.sync_copy(zb.at[pl.ds(0, sz)],
                            ac_sh.at[pl.ds(zoff + off, sz)])

    @pl.when(s == NSUB - 1)
    def _():
        for off, sz in _chunks(ZLAST):
            pltpu.sync_copy(zb.at[pl.ds(0, sz)],
                            aw_sh.at[pl.ds(zoff + off, sz)])
            pltpu.sync_copy(zb.at[pl.ds(0, sz)],
                            ac_sh.at[pl.ds(zoff + off, sz)])

    pltpu.sync_copy(tgt_hbm.at[tid], idx3)
    pltpu.sync_copy(ea_hbm.at[tid], eav)
    for l in range(8):
        ones_v[pl.ds(l * 16, 16)] = jnp.ones((16,), jnp.float32)

    @pl.when(c == 1)
    def _():
        def sub(j, carry):
            for l in range(8):
                sl = (j, pl.ds(l * 16, 16))
                idx3[sl] = idx3[sl] - HWORDS
            return carry
        lax.fori_loop(0, ECH, sub, 0)

    plsc.subcore_barrier()

    for j in range(ECH):
        pltpu.sync_copy(eav.at[j], aw_sh.at[idx3.at[j]], add=True)
        pltpu.sync_copy(ones_v, ac_sh.at[idx3.at[j]], add=True)

    plsc.subcore_barrier()

    woff = c * HWORDS + zoff

    @pl.when(s < NSUB - 1)
    def _():
        for off, sz in _chunks(ZCH):
            pltpu.sync_copy(aw_sh.at[pl.ds(zoff + off, sz)],
                            zb.at[pl.ds(0, sz)])
            pltpu.sync_copy(zb.at[pl.ds(0, sz)],
                            aw_hbm.at[pl.ds(woff + off, sz)])
            pltpu.sync_copy(ac_sh.at[pl.ds(zoff + off, sz)],
                            zb.at[pl.ds(0, sz)])
            pltpu.sync_copy(zb.at[pl.ds(0, sz)],
                            ac_hbm.at[pl.ds(woff + off, sz)])

    @pl.when(s == NSUB - 1)
    def _():
        for off, sz in _chunks(ZLAST):
            pltpu.sync_copy(aw_sh.at[pl.ds(zoff + off, sz)],
                            zb.at[pl.ds(0, sz)])
            pltpu.sync_copy(zb.at[pl.ds(0, sz)],
                            aw_hbm.at[pl.ds(woff + off, sz)])
            pltpu.sync_copy(ac_sh.at[pl.ds(zoff + off, sz)],
                            zb.at[pl.ds(0, sz)])
            pltpu.sync_copy(zb.at[pl.ds(0, sz)],
                            ac_hbm.at[pl.ds(woff + off, sz)])


_sc_build = pl.kernel(
    _sc_body,
    out_type=(jax.ShapeDtypeStruct((NCORE * HWORDS,), jnp.float32),
              jax.ShapeDtypeStruct((NCORE * HWORDS,), jnp.float32)),
    mesh=plsc.VectorSubcoreMesh(core_axis_name="c", subcore_axis_name="s"),
    scratch_types=[
        pltpu.VMEM((ECH, 128), jnp.int32),
        pltpu.VMEM((ECH, 128), jnp.float32),
        pltpu.VMEM((128,), jnp.float32),
        pltpu.VMEM((ET,), jnp.float32),
        pltpu.VMEM_SHARED((HWORDS + PADW,), jnp.float32),
        pltpu.VMEM_SHARED((HWORDS + PADW,), jnp.float32),
    ],
)


def _dotT(a, b):
    return lax.dot_general(a, b, (((0,), (0,)), ((), ())),
                           preferred_element_type=jnp.float32, precision=_HI)


def _mm(a, b):
    return lax.dot_general(a, b, (((1,), (0,)), ((), ())),
                           preferred_element_type=jnp.float32, precision=_HI)


def _tc_body(aw_ref, ac_ref, xf_ref, Wa, Ws, Wc, Wbb, Wcb, Wi, Wo, out_ref):
    Wio = _mm(Wi[...], Wo[...])
    for j in range(G):
        A_w = aw_ref[j]
        A_c = ac_ref[j]
        x = _mm(xf_ref[j], Wa[...])
        m = _mm(A_w, x)
        y = x + m
        sc = _mm(y, Ws[...])
        e = jnp.exp(sc - jnp.max(sc, axis=-1, keepdims=True))
        p = e / jnp.sum(e, axis=-1, keepdims=True)
        psq = p * p
        u = jnp.concatenate([x * psq[:, k:k + 1] for k in range(K)], axis=1)
        t = _mm(A_c, u)
        cents = []
        for k in range(K):
            v = x + t[:, k * D:(k + 1) * D]
            z = jnp.maximum(_mm(v, Wc[...]), 0.0)
            cents.append(_dotT(p[:, k:k + 1], z))
        cent = jnp.concatenate(cents, axis=0)
        m_cb = _mm(p, _mm(cent, Wcb[...]))
        be = jnp.maximum(_mm(y, Wbb[...]) + m_cb, 0.0)
        ge = jnp.sum(be, axis=0, keepdims=True) * (1.0 / NP)
        o = _mm(ge, Wio)
        out_ref[j] = jnp.broadcast_to(o, (1, D))


@jax.jit
def _run(tgt_pad, ea_pad, xf, Wa, Ws, Wc, Wbb, Wcb, Wi, Wo):
    zeros = jnp.zeros((ET,), jnp.float32)
    aw_flat, ac_flat = _sc_build(tgt_pad, ea_pad, zeros)
    aw = aw_flat.reshape(B, NP, NP)
    ac = ac_flat.reshape(B, NP, NP)
    full = lambda i: (0, 0)
    out = pl.pallas_call(
        _tc_body,
        grid=(NB,),
        in_specs=[
            pl.BlockSpec((G, NP, NP), lambda i: (i, 0, 0)),
            pl.BlockSpec((G, NP, NP), lambda i: (i, 0, 0)),
            pl.BlockSpec((G, NP, D), lambda i: (i, 0, 0)),
            pl.BlockSpec((D, D), full),
            pl.BlockSpec((D, K), full),
            pl.BlockSpec((D, D), full),
            pl.BlockSpec((D, D), full),
            pl.BlockSpec((D, D), full),
            pl.BlockSpec((D, D), full),
            pl.BlockSpec((D, 1), full),
        ],
        out_specs=pl.BlockSpec((G, 1, D), lambda i: (i, 0, 0)),
        out_shape=jax.ShapeDtypeStruct((B, 1, D), jnp.float32),
    )(aw, ac, xf, Wa, Ws, Wc, Wbb, Wcb, Wi, Wo)
    return out[:, 0, :1]


def kernel(x_feat, edge_attr, W_atom, W_score, W_c, W_bb, W_cb, W_bc, W_cc,
           W_inter, W_out, edge_index):
    del W_bc, W_cc
    goff = jnp.repeat(jnp.arange(B, dtype=jnp.int32) * NP, EP)
    tgt = edge_index[1] * NP + edge_index[0] - goff
    halves_t, halves_e = [], []
    for c in range(NCORE):
        th = tgt[c * EH:(c + 1) * EH].reshape(NSUB, EH // NSUB)
        eh = edge_attr[c * EH:(c + 1) * EH].reshape(NSUB, EH // NSUB)
        pad = ET - EH // NSUB
        halves_t.append(jnp.pad(th, ((0, 0), (0, pad)),
                                constant_values=(c + 1) * HWORDS))
        halves_e.append(jnp.pad(eh, ((0, 0), (0, pad))))
    tgt_pad = jnp.concatenate(halves_t, 0).reshape(NW, ECH, 128)
    ea_pad = jnp.concatenate(halves_e, 0).reshape(NW, ECH, 128)
    xf = x_feat.reshape(B, NP, D)
    return _run(tgt_pad, ea_pad, xf, W_atom, W_score, W_c, W_bb, W_cb,
                W_inter, W_out)


# speedup vs baseline: 31.5583x; 2.0965x over previous
import jax
import jax.numpy as jnp
from jax import lax
from jax.experimental import pallas as pl
from jax.experimental.pallas import tpu as pltpu
from jax.experimental.pallas import tpu_sc as plsc

N = 10000
B = 100
NP = 100
E = 160000
EP = 1600
K = 4
D = 128
NPP = 128
G = 20
NB = B // G
GR = G * NPP
GB = 32

NCORE = 2
NSUB = 16
NW = NCORE * NSUB
HG = B // NCORE
HWORDS = HG * NPP * NPP
EH = E // NCORE
ET = 5120
ECH = ET // 128
ZCH = HWORDS // NSUB

_HI = lax.Precision.HIGHEST


def _sc_body(tgt_hbm, ea_hbm, zeros_hbm, aw_hbm, ac_hbm,
             idx3, eav, ones_v, zb, aw_sh, ac_sh):
    c = lax.axis_index("c")
    s = lax.axis_index("s")
    tid = c * NSUB + s

    pltpu.sync_copy(zeros_hbm, zb)
    zoff = s * ZCH
    for q in range(ZCH // ET):
        pltpu.sync_copy(zb, aw_sh.at[pl.ds(zoff + q * ET, ET)])
        pltpu.sync_copy(zb, ac_sh.at[pl.ds(zoff + q * ET, ET)])

    pltpu.sync_copy(tgt_hbm.at[tid], idx3)
    pltpu.sync_copy(ea_hbm.at[tid], eav)
    for l in range(8):
        ones_v[pl.ds(l * 16, 16)] = jnp.ones((16,), jnp.float32)

    @pl.when(c == 1)
    def _():
        def sub(j, carry):
            for l in range(8):
                sl = (j, pl.ds(l * 16, 16))
                idx3[sl] = idx3[sl] - HWORDS
            return carry
        lax.fori_loop(0, ECH, sub, 0)

    plsc.subcore_barrier()

    for j in range(ECH):
        pltpu.sync_copy(eav.at[j], aw_sh.at[idx3.at[j]], add=True)
        pltpu.sync_copy(ones_v, ac_sh.at[idx3.at[j]], add=True)

    plsc.subcore_barrier()

    woff = c * HWORDS + zoff
    for q in range(ZCH // ET):
        pltpu.sync_copy(aw_sh.at[pl.ds(zoff + q * ET, ET)], zb)
        pltpu.sync_copy(zb, aw_hbm.at[pl.ds(woff + q * ET, ET)])
        pltpu.sync_copy(ac_sh.at[pl.ds(zoff + q * ET, ET)], zb)
        pltpu.sync_copy(zb, ac_hbm.at[pl.ds(woff + q * ET, ET)])


_sc_build = pl.kernel(
    _sc_body,
    out_type=(jax.ShapeDtypeStruct((NCORE * HWORDS,), jnp.float32),
              jax.ShapeDtypeStruct((NCORE * HWORDS,), jnp.float32)),
    mesh=plsc.VectorSubcoreMesh(core_axis_name="c", subcore_axis_name="s"),
    scratch_types=[
        pltpu.VMEM((ECH, 128), jnp.int32),
        pltpu.VMEM((ECH, 128), jnp.float32),
        pltpu.VMEM((128,), jnp.float32),
        pltpu.VMEM((ET,), jnp.float32),
        pltpu.VMEM_SHARED((HWORDS + 8,), jnp.float32),
        pltpu.VMEM_SHARED((HWORDS + 8,), jnp.float32),
    ],
)


def _dotT(a, b):
    return lax.dot_general(a, b, (((0,), (0,)), ((), ())),
                           preferred_element_type=jnp.float32, precision=_HI)


def _mm(a, b):
    return lax.dot_general(a, b, (((1,), (0,)), ((), ())),
                           preferred_element_type=jnp.float32, precision=_HI)


def _tc_body(aw_ref, ac_ref, xf_ref, Wa, Ws, Wc, Wbb, Wcb, Wi, Wo, out_ref,
             ys, ts):
    Wio = _mm(Wi[...], Wo[...])
    X = _mm(xf_ref[...], Wa[...])
    for j in range(G):
        r = pl.ds(j * NPP, NPP)
        ys[r] = X[j * NPP:(j + 1) * NPP] + _mm(aw_ref[r], X[j * NPP:(j + 1) * NPP])
    Y = ys[...]
    S = _mm(Y, Ws[...])
    e = jnp.exp(S - jnp.max(S, axis=-1, keepdims=True))
    P = e / jnp.sum(e, axis=-1, keepdims=True)
    rows = lax.broadcasted_iota(jnp.int32, (GR, 1), 0)
    P = P * ((rows & (NPP - 1)) < NP).astype(jnp.float32)
    U = jnp.concatenate([X * (P[:, k:k + 1] * P[:, k:k + 1])
                         for k in range(K)], axis=1)
    for j in range(G):
        r = pl.ds(j * NPP, NPP)
        ts[r] = _mm(ac_ref[r], U[j * NPP:(j + 1) * NPP])
    T = ts[...]
    Gm = (lax.broadcasted_iota(jnp.int32, (GB, GR), 1) // NPP
          == lax.broadcasted_iota(jnp.int32, (GB, GR), 0)).astype(jnp.float32)
    GmT = (lax.broadcasted_iota(jnp.int32, (GR, GB), 0) // NPP
           == lax.broadcasted_iota(jnp.int32, (GR, GB), 1)).astype(jnp.float32)
    Mcb = jnp.zeros((GR, D), jnp.float32)
    for k in range(K):
        Zk = jnp.maximum(_mm(X + T[:, k * D:(k + 1) * D], Wc[...]), 0.0)
        Qk = P[:, k:k + 1] * Zk
        Dk = _mm(_mm(Gm, Qk), Wcb[...])
        Mcb = Mcb + P[:, k:k + 1] * _mm(GmT, Dk)
    be = jnp.maximum(_mm(Y, Wbb[...]) + Mcb, 0.0)
    ge = _mm(Gm, be) * (1.0 / NP)
    o = _mm(ge, Wio)
    out_ref[...] = jnp.broadcast_to(o[:G].reshape(G, 1, 1), (G, 1, D))


@jax.jit
def _run(tgt_pad, ea_pad, xfp, Wa, Ws, Wc, Wbb, Wcb, Wi, Wo):
    zeros = jnp.zeros((ET,), jnp.float32)
    aw_flat, ac_flat = _sc_build(tgt_pad, ea_pad, zeros)
    aw = aw_flat.reshape(B * NPP, NPP)
    ac = ac_flat.reshape(B * NPP, NPP)
    full = lambda i: (0, 0)
    out = pl.pallas_call(
        _tc_body,
        grid=(NB,),
        in_specs=[
            pl.BlockSpec((GR, NPP), lambda i: (i, 0)),
            pl.BlockSpec((GR, NPP), lambda i: (i, 0)),
            pl.BlockSpec((GR, D), lambda i: (i, 0)),
            pl.BlockSpec((D, D), full),
            pl.BlockSpec((D, K), full),
            pl.BlockSpec((D, D), full),
            pl.BlockSpec((D, D), full),
            pl.BlockSpec((D, D), full),
            pl.BlockSpec((D, D), full),
            pl.BlockSpec((D, 1), full),
        ],
        out_specs=pl.BlockSpec((G, 1, D), lambda i: (i, 0, 0)),
        out_shape=jax.ShapeDtypeStruct((B, 1, D), jnp.float32),
        scratch_shapes=[
            pltpu.VMEM((GR, D), jnp.float32),
            pltpu.VMEM((GR, K * D), jnp.float32),
        ],
    )(aw, ac, xfp, Wa, Ws, Wc, Wbb, Wcb, Wi, Wo)
    return out[:, 0, :1]


def kernel(x_feat, edge_attr, W_atom, W_score, W_c, W_bb, W_cb, W_bc, W_cc,
           W_inter, W_out, edge_index):
    del W_bc, W_cc
    goff = jnp.repeat(jnp.arange(B, dtype=jnp.int32) * NP, EP)
    gpad = jnp.repeat(jnp.arange(B, dtype=jnp.int32) * (NPP * NPP), EP)
    tgt = (edge_index[1] - goff) * NPP + (edge_index[0] - goff) + gpad
    halves_t, halves_e = [], []
    for c in range(NCORE):
        th = tgt[c * EH:(c + 1) * EH].reshape(NSUB, EH // NSUB)
        eh = edge_attr[c * EH:(c + 1) * EH].reshape(NSUB, EH // NSUB)
        pad = ET - EH // NSUB
        halves_t.append(jnp.pad(th, ((0, 0), (0, pad)),
                                constant_values=(c + 1) * HWORDS))
        halves_e.append(jnp.pad(eh, ((0, 0), (0, pad))))
    tgt_pad = jnp.concatenate(halves_t, 0).reshape(NW, ECH, 128)
    ea_pad = jnp.concatenate(halves_e, 0).reshape(NW, ECH, 128)
    xfp = jnp.pad(x_feat.reshape(B, NP, D),
                  ((0, 0), (0, NPP - NP), (0, 0))).reshape(B * NPP, D)
    return _run(tgt_pad, ea_pad, xfp, W_atom, W_score, W_c, W_bb, W_cb,
                W_inter, W_out)


# speedup vs baseline: 40.0013x; 1.2675x over previous
import jax
import jax.numpy as jnp
from jax import lax
from jax.experimental import pallas as pl
from jax.experimental.pallas import tpu as pltpu
from jax.experimental.pallas import tpu_sc as plsc

N = 10000
B = 100
NP = 100
E = 160000
EP = 1600
K = 4
D = 128
NPP = 128
G = 20
NB = B // G
GR = G * NPP
GB = 32

NCORE = 2
NSUB = 16
NW = NCORE * NSUB
HG = B // NCORE
HWORDS = HG * NPP * NPP
EH = E // NCORE
ET = 5120
ECH = ET // 128
ZCH = HWORDS // NSUB

_HI = lax.Precision.HIGHEST


def _sc_body(tgt_hbm, ea_hbm, zeros_hbm, aw_hbm, ac_hbm,
             idx3, eav, ones_v, zb, aw_sh, ac_sh):
    c = lax.axis_index("c")
    s = lax.axis_index("s")
    tid = c * NSUB + s

    pltpu.sync_copy(zeros_hbm, zb)
    zoff = s * ZCH
    for q in range(ZCH // ET):
        pltpu.sync_copy(zb, aw_sh.at[pl.ds(zoff + q * ET, ET)])
        pltpu.sync_copy(zb, ac_sh.at[pl.ds(zoff + q * ET, ET)])

    pltpu.sync_copy(tgt_hbm.at[tid], idx3)
    pltpu.sync_copy(ea_hbm.at[tid], eav)
    for l in range(8):
        ones_v[pl.ds(l * 16, 16)] = jnp.ones((16,), jnp.float32)

    @pl.when(c == 1)
    def _():
        def sub(j, carry):
            for l in range(8):
                sl = (j, pl.ds(l * 16, 16))
                idx3[sl] = idx3[sl] - HWORDS
            return carry
        lax.fori_loop(0, ECH, sub, 0)

    plsc.subcore_barrier()

    for j in range(ECH):
        pltpu.sync_copy(eav.at[j], aw_sh.at[idx3.at[j]], add=True)
        pltpu.sync_copy(ones_v, ac_sh.at[idx3.at[j]], add=True)

    plsc.subcore_barrier()

    woff = c * HWORDS + zoff
    for q in range(ZCH // ET):
        pltpu.sync_copy(aw_sh.at[pl.ds(zoff + q * ET, ET)], zb)
        pltpu.sync_copy(zb, aw_hbm.at[pl.ds(woff + q * ET, ET)])
        pltpu.sync_copy(ac_sh.at[pl.ds(zoff + q * ET, ET)], zb)
        pltpu.sync_copy(zb, ac_hbm.at[pl.ds(woff + q * ET, ET)])


_sc_build = pl.kernel(
    _sc_body,
    out_type=(jax.ShapeDtypeStruct((NCORE * HWORDS,), jnp.float32),
              jax.ShapeDtypeStruct((NCORE * HWORDS,), jnp.float32)),
    mesh=plsc.VectorSubcoreMesh(core_axis_name="c", subcore_axis_name="s"),
    scratch_types=[
        pltpu.VMEM((ECH, 128), jnp.int32),
        pltpu.VMEM((ECH, 128), jnp.float32),
        pltpu.VMEM((128,), jnp.float32),
        pltpu.VMEM((ET,), jnp.float32),
        pltpu.VMEM_SHARED((HWORDS + 8,), jnp.float32),
        pltpu.VMEM_SHARED((HWORDS + 8,), jnp.float32),
    ],
)


def _dotT(a, b):
    return lax.dot_general(a, b, (((0,), (0,)), ((), ())),
                           preferred_element_type=jnp.float32, precision=_HI)


def _mm(a, b):
    return lax.dot_general(a, b, (((1,), (0,)), ((), ())),
                           preferred_element_type=jnp.float32, precision=_HI)


def _mmd(a, b):
    return lax.dot_general(a, b, (((1,), (0,)), ((), ())),
                           preferred_element_type=jnp.float32)


def _tc_body(aw_ref, ac_ref, xf_ref, Wa, Ws, Wc, Wbb, Wcb, Wi, Wo, out_ref,
             ys, ts):
    X = _mmd(xf_ref[...], Wa[...])
    for j in range(G):
        r = pl.ds(j * NPP, NPP)
        ys[r] = X[j * NPP:(j + 1) * NPP] + _mm(aw_ref[r], X[j * NPP:(j + 1) * NPP])
    Y = ys[...]
    S = _mmd(Y, Ws[...])
    e = jnp.exp(S - jnp.max(S, axis=-1, keepdims=True))
    P = e / jnp.sum(e, axis=-1, keepdims=True)
    rows = lax.broadcasted_iota(jnp.int32, (GR, 1), 0)
    P = P * ((rows & (NPP - 1)) < NP).astype(jnp.float32)
    U = jnp.concatenate([X * (P[:, k:k + 1] * P[:, k:k + 1])
                         for k in range(K)], axis=1)
    for j in range(G):
        r = pl.ds(j * NPP, NPP)
        ts[r] = _mm(ac_ref[r], U[j * NPP:(j + 1) * NPP])
    T = ts[...]
    Gm = (lax.broadcasted_iota(jnp.int32, (GB, GR), 1) // NPP
          == lax.broadcasted_iota(jnp.int32, (GB, GR), 0)).astype(jnp.float32)
    GmT = (lax.broadcasted_iota(jnp.int32, (GR, GB), 0) // NPP
           == lax.broadcasted_iota(jnp.int32, (GR, GB), 1)).astype(jnp.float32)
    Pre = jnp.zeros((GR, D), jnp.float32)
    for k in range(K):
        Hk = P[:, k:k + 1] * (X + T[:, k * D:(k + 1) * D])
        Zk = jnp.maximum(_mmd(Hk, Wc[...]), 0.0)
        Ck = _mm(Gm, Zk)
        Pre = Pre + P[:, k:k + 1] * _mm(GmT, Ck)
    Mcb = _mmd(Pre, Wcb[...])
    be = jnp.maximum(_mmd(Y, Wbb[...]) + Mcb, 0.0)
    NE = _mmd(be, Wi[...])
    ge = _mm(Gm, NE) * (1.0 / NP)
    o = _mmd(ge, Wo[...])
    out_ref[...] = jnp.broadcast_to(o[:G].reshape(G, 1, 1), (G, 1, D))


@jax.jit
def _run(tgt_pad, ea_pad, xfp, Wa, Ws, Wc, Wbb, Wcb, Wi, Wo):
    zeros = jnp.zeros((ET,), jnp.float32)
    aw_flat, ac_flat = _sc_build(tgt_pad, ea_pad, zeros)
    aw = aw_flat.reshape(B * NPP, NPP)
    ac = ac_flat.reshape(B * NPP, NPP)
    full = lambda i: (0, 0)
    out = pl.pallas_call(
        _tc_body,
        grid=(NB,),
        in_specs=[
            pl.BlockSpec((GR, NPP), lambda i: (i, 0)),
            pl.BlockSpec((GR, NPP), lambda i: (i, 0)),
            pl.BlockSpec((GR, D), lambda i: (i, 0)),
            pl.BlockSpec((D, D), full),
            pl.BlockSpec((D, K), full),
            pl.BlockSpec((D, D), full),
            pl.BlockSpec((D, D), full),
            pl.BlockSpec((D, D), full),
            pl.BlockSpec((D, D), full),
            pl.BlockSpec((D, 1), full),
        ],
        out_specs=pl.BlockSpec((G, 1, D), lambda i: (i, 0, 0)),
        out_shape=jax.ShapeDtypeStruct((B, 1, D), jnp.float32),
        scratch_shapes=[
            pltpu.VMEM((GR, D), jnp.float32),
            pltpu.VMEM((GR, K * D), jnp.float32),
        ],
    )(aw, ac, xfp, Wa, Ws, Wc, Wbb, Wcb, Wi, Wo)
    return out[:, 0, :1]


def kernel(x_feat, edge_attr, W_atom, W_score, W_c, W_bb, W_cb, W_bc, W_cc,
           W_inter, W_out, edge_index):
    del W_bc, W_cc
    goff = jnp.repeat(jnp.arange(B, dtype=jnp.int32) * NP, EP)
    gpad = jnp.repeat(jnp.arange(B, dtype=jnp.int32) * (NPP * NPP), EP)
    tgt = (edge_index[1] - goff) * NPP + (edge_index[0] - goff) + gpad
    halves_t, halves_e = [], []
    for c in range(NCORE):
        th = tgt[c * EH:(c + 1) * EH].reshape(NSUB, EH // NSUB)
        eh = edge_attr[c * EH:(c + 1) * EH].reshape(NSUB, EH // NSUB)
        pad = ET - EH // NSUB
        halves_t.append(jnp.pad(th, ((0, 0), (0, pad)),
                                constant_values=(c + 1) * HWORDS))
        halves_e.append(jnp.pad(eh, ((0, 0), (0, pad))))
    tgt_pad = jnp.concatenate(halves_t, 0).reshape(NW, ECH, 128)
    ea_pad = jnp.concatenate(halves_e, 0).reshape(NW, ECH, 128)
    xfp = jnp.pad(x_feat.reshape(B, NP, D),
                  ((0, 0), (0, NPP - NP), (0, 0))).reshape(B * NPP, D)
    return _run(tgt_pad, ea_pad, xfp, W_atom, W_score, W_c, W_bb, W_cb,
                W_inter, W_out)
